# baseline (speedup 1.0000x reference)
import jax
import jax.numpy as jnp
from jax import lax
from jax.experimental import pallas as pl
from jax.experimental.pallas import tpu as pltpu
from jax.experimental.pallas import tpu_sc as plsc

N_VOCAB = 100000
N_EMBED = 64
BATCH = 16384
N_SAMPLES = 5

NC = 2
NS = 16
NW = NC * NS
DIMS_PER_W = N_EMBED // NW

TOTAL = BATCH * (2 + N_SAMPLES)
N_BC = BATCH * (1 + N_SAMPLES)

IC = 4096
UNROLL = 16


def _gather_body(in_idx_hbm, bc_idx_hbm, in_tab_t, out_tab_t, out_t,
                 row_v, idx_vs, out_vs, isems, wsems):
    wid = lax.axis_index("s") * NC + lax.axis_index("c")

    def gather_chunk(idx_v, out_v):
        @plsc.parallel_loop(0, IC, 16, unroll=UNROLL)
        def body(i):
            iv = idx_v[pl.ds(i, 16)]
            out_v[pl.ds(i, 16)] = plsc.load_gather(row_v, [iv])

    def do_dim(j, tab, idx_hbm, idx_n, out_off):
        pltpu.sync_copy(tab.at[j], row_v)
        nch = idx_n // IC
        ids = [None, None]
        wds = [None] * nch
        ids[0] = pltpu.async_copy(idx_hbm.at[pl.ds(0, IC)], idx_vs[0],
                                  isems[0])
        for c in range(nch):
            b = c % 2
            if c + 1 < nch:
                ids[(c + 1) % 2] = pltpu.async_copy(
                    idx_hbm.at[pl.ds((c + 1) * IC, IC)],
                    idx_vs[(c + 1) % 2], isems[(c + 1) % 2])
            ids[b].wait()
            if c - 2 >= 0:
                wds[c - 2].wait()
            gather_chunk(idx_vs[b], out_vs[b])
            wds[c] = pltpu.async_copy(
                out_vs[b], out_t.at[j, pl.ds(out_off + c * IC, IC)],
                wsems[b])
        for c in range(max(0, nch - 2), nch):
            wds[c].wait()

    for t in range(DIMS_PER_W):
        j = wid * DIMS_PER_W + t
        do_dim(j, in_tab_t, in_idx_hbm, BATCH, 0)
        do_dim(j, out_tab_t, bc_idx_hbm, N_BC, BATCH)


def kernel(input_words, output_words, noise_words, in_embed_weight,
           out_embed_weight):
    bc_idx = jnp.concatenate(
        [output_words.astype(jnp.int32),
         noise_words.reshape(-1).astype(jnp.int32)], axis=0)
    mesh = plsc.VectorSubcoreMesh(core_axis_name="c", subcore_axis_name="s")
    f = pl.kernel(
        _gather_body,
        mesh=mesh,
        out_type=jax.ShapeDtypeStruct((N_EMBED, TOTAL), jnp.float32),
        scratch_types=[
            pltpu.VMEM((N_VOCAB,), jnp.float32),
            [pltpu.VMEM((IC,), jnp.int32)] * 2,
            [pltpu.VMEM((IC,), jnp.float32)] * 2,
            [pltpu.SemaphoreType.DMA] * 2,
            [pltpu.SemaphoreType.DMA] * 2,
        ],
        compiler_params=pltpu.CompilerParams(use_tc_tiling_on_sc=True,
                                             needs_layout_passes=False),
    )
    out_t = f(
        input_words.astype(jnp.int32),
        bc_idx,
        in_embed_weight.T,
        out_embed_weight.T,
    )
    return out_t.T


# speedup vs baseline: 1.0209x; 1.0209x over previous
import jax
import jax.numpy as jnp
from jax import lax
from jax.experimental import pallas as pl
from jax.experimental.pallas import tpu as pltpu
from jax.experimental.pallas import tpu_sc as plsc

N_VOCAB = 100000
N_EMBED = 64
BATCH = 16384
N_SAMPLES = 5

NC = 2
NS = 16
NW = NC * NS
DIMS_PER_W = N_EMBED // NW

TOTAL = BATCH * (2 + N_SAMPLES)
N_BC = BATCH * (1 + N_SAMPLES)

IC = 4096
UNROLL = 8


def _gather_body(in_idx_hbm, bc_idx_hbm, in_tab_t, out_tab_t, out_t,
                 row_v, idx_vs, out_vs, isems, wsems):
    wid = lax.axis_index("s") * NC + lax.axis_index("c")

    def gather_chunk(idx_v, out_v):
        @plsc.parallel_loop(0, IC, 16, unroll=UNROLL)
        def body(i):
            iv = idx_v[pl.ds(i, 16)]
            out_v[pl.ds(i, 16)] = plsc.load_gather(row_v, [iv])

    passes = []
    for t in range(DIMS_PER_W):
        j = wid * DIMS_PER_W + t
        passes.append((j, in_tab_t, in_idx_hbm, BATCH, 0))
        passes.append((j, out_tab_t, bc_idx_hbm, N_BC, BATCH))

    def first_idx_copy(p):
        _, _, idx_hbm, _, _ = passes[p]
        return pltpu.async_copy(idx_hbm.at[pl.ds(0, IC)], idx_vs[0],
                                isems[0])

    nxt_first = first_idx_copy(0)
    for p, (j, tab, idx_hbm, idx_n, out_off) in enumerate(passes):
        pltpu.sync_copy(tab.at[j], row_v)
        nch = idx_n // IC
        ids = [nxt_first, None]
        wds = [None] * nch
        nxt_first = None
        for c in range(nch):
            b = c % 2
            if c + 1 < nch:
                ids[(c + 1) % 2] = pltpu.async_copy(
                    idx_hbm.at[pl.ds((c + 1) * IC, IC)],
                    idx_vs[(c + 1) % 2], isems[(c + 1) % 2])
            ids[b].wait()
            if c - 2 >= 0:
                wds[c - 2].wait()
            gather_chunk(idx_vs[b], out_vs[b])
            if c == nch - 1 and p + 1 < len(passes):
                nxt_first = first_idx_copy(p + 1)
            wds[c] = pltpu.async_copy(
                out_vs[b], out_t.at[j, pl.ds(out_off + c * IC, IC)],
                wsems[b])
        for c in range(max(0, nch - 2), nch):
            wds[c].wait()


def kernel(input_words, output_words, noise_words, in_embed_weight,
           out_embed_weight):
    bc_idx = jnp.concatenate(
        [output_words.astype(jnp.int32),
         noise_words.reshape(-1).astype(jnp.int32)], axis=0)
    mesh = plsc.VectorSubcoreMesh(core_axis_name="c", subcore_axis_name="s")
    f = pl.kernel(
        _gather_body,
        mesh=mesh,
        out_type=jax.ShapeDtypeStruct((N_EMBED, TOTAL), jnp.float32),
        scratch_types=[
            pltpu.VMEM((N_VOCAB,), jnp.float32),
            [pltpu.VMEM((IC,), jnp.int32)] * 2,
            [pltpu.VMEM((IC,), jnp.float32)] * 2,
            [pltpu.SemaphoreType.DMA] * 2,
            [pltpu.SemaphoreType.DMA] * 2,
        ],
        compiler_params=pltpu.CompilerParams(use_tc_tiling_on_sc=True,
                                             needs_layout_passes=False),
    )
    out_t = f(
        input_words.astype(jnp.int32),
        bc_idx,
        in_embed_weight.T,
        out_embed_weight.T,
    )
    return out_t.T
